# baseline (speedup 1.0000x reference)
import jax
import jax.numpy as jnp
from jax import lax
from jax.experimental import pallas as pl
from jax.experimental.pallas import tpu as pltpu
from jax.experimental.pallas import tpu_sc as plsc

DIM = 32
NC = 2
NS = 16
NW = NC * NS
L = 16

GATHER = 128
GPG = 10
GROUP = GATHER * GPG


def _emb_body(table_hbm, xt_hbm, out_hbm, idx_v, idx2_v, rows_v,
              semg0, semg1, semo0, semo1):
    seq = xt_hbm.shape[0]
    bpw = xt_hbm.shape[1] // NW
    rows_per_w = seq * bpw
    n_groups = rows_per_w // GROUP
    wid = lax.axis_index("s") * NC + lax.axis_index("c")
    b0 = wid * bpw

    pltpu.sync_copy(xt_hbm.at[:, pl.ds(b0, bpw)], idx_v)

    sems_g = (semg0, semg1)
    sems_o = (semo0, semo1)

    def make_idx(g, buf):
        def step(j, carry):
            for u in range(8):
                k = g * GROUP + (j * 8 + u) * L + lax.iota(jnp.int32, L)
                s = lax.rem(k, seq)
                bb = lax.div(k, seq)
                idx2_v[buf, pl.ds((j * 8 + u) * L, L)] = plsc.load_gather(
                    idx_v, [s, bb])
            return carry
        lax.fori_loop(0, GROUP // (8 * L), step, 0)

    def fire(g, buf):
        for j in range(GPG):
            pltpu.async_copy(
                table_hbm.at[idx2_v.at[buf, pl.ds(j * GATHER, GATHER)]],
                rows_v.at[buf, pl.ds(j * GATHER, GATHER)],
                sems_g[buf],
            )

    def drain_gathers(buf):
        pltpu.make_async_copy(
            table_hbm.at[pl.ds(0, GROUP)], rows_v.at[buf], sems_g[buf]
        ).wait()

    def out_slice(g):
        return out_hbm.at[pl.ds(wid * rows_per_w + g * GROUP, GROUP)]

    def wait_writeout(buf, g):
        pltpu.make_async_copy(rows_v.at[buf], out_slice(g), sems_o[buf]).wait()

    make_idx(0, 0)

    def pair(p, carry):
        g0 = 2 * p
        g1 = 2 * p + 1

        @pl.when(p >= 1)
        def _():
            wait_writeout(0, g0 - 2)

        fire(g0, 0)
        make_idx(g1, 1)

        @pl.when(p >= 1)
        def _():
            wait_writeout(1, g1 - 2)

        drain_gathers(0)
        pltpu.async_copy(rows_v.at[0], out_slice(g0), sems_o[0])

        fire(g1, 1)

        @pl.when(p + 1 < n_groups // 2)
        def _():
            make_idx(g0 + 2, 0)

        drain_gathers(1)
        pltpu.async_copy(rows_v.at[1], out_slice(g1), sems_o[1])
        return carry

    lax.fori_loop(0, n_groups // 2, pair, 0)
    wait_writeout(0, n_groups - 2)
    wait_writeout(1, n_groups - 1)


def _make_call(seq, bpw):
    total = seq * bpw * NW
    return pl.kernel(
        _emb_body,
        out_type=jax.ShapeDtypeStruct((total, DIM), jnp.float32),
        mesh=plsc.VectorSubcoreMesh(core_axis_name="c", subcore_axis_name="s"),
        scratch_types=[
            pltpu.VMEM((seq, bpw), jnp.int32),
            pltpu.VMEM((2, GROUP), jnp.int32),
            pltpu.VMEM((2, GROUP, DIM), jnp.float32),
            pltpu.SemaphoreType.DMA,
            pltpu.SemaphoreType.DMA,
            pltpu.SemaphoreType.DMA,
            pltpu.SemaphoreType.DMA,
        ],
        compiler_params=pltpu.CompilerParams(
            use_tc_tiling_on_sc=False, needs_layout_passes=False),
    )


def kernel(x, table):
    b, seq = x.shape
    bpw = b // NW
    assert b % NW == 0 and (seq * bpw) % (2 * GROUP) == 0
    xt = x.T.astype(jnp.int32)
    out = _make_call(seq, bpw)(table, xt)
    return out.reshape(b, seq, DIM)


# speedup vs baseline: 1.4113x; 1.4113x over previous
import jax
import jax.numpy as jnp
from jax import lax
from jax.experimental import pallas as pl
from jax.experimental.pallas import tpu as pltpu
from jax.experimental.pallas import tpu_sc as plsc

DIM = 32
NC = 2
NS = 16
NW = NC * NS
L = 16
GATHER = 128


def _gather_body(tab_hbm, xt_hbm, out_hbm, idxf_v, srow_v, scol_v, rows_v,
                 tile_v, semi, semg0, semg1, semo0, semo1):
    seq = xt_hbm.shape[0]
    bpw = xt_hbm.shape[1] // NW
    upw = seq * (bpw // GATHER)
    wid = lax.axis_index("s") * NC + lax.axis_index("c")
    b0 = wid * bpw

    for s in range(seq):
        pltpu.async_copy(xt_hbm.at[s, pl.ds(b0, bpw)],
                         idxf_v.at[pl.ds(s * bpw, bpw)], semi)
    for s in range(seq):
        pltpu.make_async_copy(xt_hbm.at[s, pl.ds(b0, bpw)],
                              idxf_v.at[pl.ds(s * bpw, bpw)], semi).wait()

    sems_g = (semg0, semg1)
    sems_o = (semo0, semo1)
    nbsub = bpw // GATHER

    def prep(u, buf):
        base = u * GATHER
        for v in range(GATHER // L):
            iv = idxf_v[pl.ds(base + v * L, L)]
            srow_v[buf, pl.ds(v * L, L)] = lax.shift_right_logical(iv, 2)
            scol_v[buf, pl.ds(v * L, L)] = (iv & 3) * DIM

    def fire(buf):
        pltpu.async_copy(tab_hbm.at[srow_v.at[buf]], rows_v.at[buf],
                         sems_g[buf])

    def drain_gather(buf):
        pltpu.make_async_copy(tab_hbm.at[pl.ds(0, GATHER)], rows_v.at[buf],
                              sems_g[buf]).wait()

    def extract(buf):
        rows = rows_v.at[buf]
        cols = [scol_v[buf, pl.ds(v * L, L)] for v in range(GATHER // L)]
        segs = [jnp.arange(v * L, v * L + L, dtype=jnp.int32)
                for v in range(GATHER // L)]
        for c in range(DIM):
            for v in range(GATHER // L):
                tile_v[buf, c, pl.ds(v * L, L)] = plsc.load_gather(
                    rows, [segs[v], cols[v] + c])

    def out_slice(u):
        s = lax.div(u, nbsub)
        bg = b0 + lax.rem(u, nbsub) * GATHER
        return out_hbm.at[s, :, pl.ds(bg, GATHER)]

    def wait_writeout(buf, u):
        pltpu.make_async_copy(tile_v.at[buf], out_slice(u), sems_o[buf]
                              ).wait()

    prep(0, 0)
    fire(0)

    def pair(p, carry):
        u0 = 2 * p
        u1 = 2 * p + 1

        prep(u1, 1)
        drain_gather(0)
        fire(1)

        @pl.when(p >= 1)
        def _():
            wait_writeout(0, u0 - 2)

        extract(0)
        pltpu.async_copy(tile_v.at[0], out_slice(u0), sems_o[0])

        @pl.when(p + 1 < upw // 2)
        def _():
            prep(u0 + 2, 0)
            fire(0)

        drain_gather(1)

        @pl.when(p >= 1)
        def _():
            wait_writeout(1, u1 - 2)

        extract(1)
        pltpu.async_copy(tile_v.at[1], out_slice(u1), sems_o[1])
        return carry

    lax.fori_loop(0, upw // 2, pair, 0)
    wait_writeout(0, upw - 2)
    wait_writeout(1, upw - 1)


def _make_gather(seq, b):
    return pl.kernel(
        _gather_body,
        out_type=jax.ShapeDtypeStruct((seq, DIM, b), jnp.float32),
        mesh=plsc.VectorSubcoreMesh(core_axis_name="c", subcore_axis_name="s"),
        scratch_types=[
            pltpu.VMEM((seq * (b // NW),), jnp.int32),
            pltpu.VMEM((2, GATHER), jnp.int32),
            pltpu.VMEM((2, GATHER), jnp.int32),
            pltpu.VMEM((2, GATHER, 128), jnp.float32),
            pltpu.VMEM((2, DIM, GATHER), jnp.float32),
            pltpu.SemaphoreType.DMA,
            pltpu.SemaphoreType.DMA,
            pltpu.SemaphoreType.DMA,
            pltpu.SemaphoreType.DMA,
            pltpu.SemaphoreType.DMA,
        ],
        compiler_params=pltpu.CompilerParams(
            use_tc_tiling_on_sc=True, needs_layout_passes=False),
    )


def kernel(x, table):
    b, seq = x.shape
    n, d = table.shape
    assert d == DIM and (n * d) % 128 == 0
    xt = x.T.astype(jnp.int32)
    tab_r = table.reshape(n * d // 128, 128)
    out = _make_gather(seq, b)(tab_r, xt)
    return jnp.transpose(out, (2, 0, 1))
